# Initial kernel scaffold; baseline (speedup 1.0000x reference)
import functools

import jax
import jax.numpy as jnp
from jax import lax
from jax.experimental import pallas as pl
from jax.experimental.pallas import tpu as pltpu
from jax.experimental.pallas import tpu_sc as plsc

_TEMP = 0.05
_INV_TEMP = 1.0 / _TEMP
_NUM_CLUSTERS = 100000
_DIM = 64
_BATCH = 1024

_C_BLK = 5000
_NB = _NUM_CLUSTERS // _C_BLK


def _make_sc_gather():
    info = plsc.get_sparse_core_info()
    nc, ns = info.num_cores, info.num_subcores
    nw = nc * ns
    b_per_w = _BATCH // nw
    mesh = plsc.VectorSubcoreMesh(core_axis_name="c", subcore_axis_name="s")

    @functools.partial(
        pl.kernel,
        mesh=mesh,
        out_type=jax.ShapeDtypeStruct((_BATCH, _DIM), jnp.float32),
        scratch_types=[
            pltpu.VMEM((b_per_w,), jnp.int32),
            pltpu.VMEM((b_per_w, _DIM), jnp.float32),
            pltpu.SemaphoreType.DMA,
        ],
    )
    def gather_rows(table_hbm, idx_hbm, out_hbm, idx_v, rows_v, sem):
        wid = lax.axis_index("s") * nc + lax.axis_index("c")
        base = wid * b_per_w
        pltpu.sync_copy(idx_hbm.at[pl.ds(base, b_per_w)], idx_v)
        pltpu.async_copy(table_hbm.at[idx_v], rows_v, sem).wait()
        pltpu.sync_copy(rows_v, out_hbm.at[pl.ds(base, b_per_w)])

    return gather_rows


_sc_gather = _make_sc_gather()


def _lse_body(f_ref, mem_ref, tgt_ref, out_ref, m_ref, s_ref):
    i = pl.program_id(0)

    @pl.when(i == 0)
    def _init():
        m_ref[...] = jnp.full((_BATCH, 1), -1e30, jnp.float32)
        s_ref[...] = jnp.zeros((_BATCH, 1), jnp.float32)

    f = f_ref[...] * _INV_TEMP
    scores = lax.dot_general(
        f, mem_ref[...],
        (((1,), (1,)), ((), ())),
        preferred_element_type=jnp.float32,
    )
    bm = jnp.max(scores, axis=1, keepdims=True)
    m_prev = m_ref[...]
    s_prev = s_ref[...]
    m_new = jnp.maximum(m_prev, bm)
    s_new = s_prev * jnp.exp(m_prev - m_new) + jnp.sum(
        jnp.exp(scores - m_new), axis=1, keepdims=True)
    m_ref[...] = m_new
    s_ref[...] = s_new

    @pl.when(i == _NB - 1)
    def _final():
        tl = jnp.sum(f * tgt_ref[...], axis=1, keepdims=True)
        nll = m_new + jnp.log(s_new) - tl
        out_ref[0, 0] = jnp.mean(nll)


_lse_call = pl.pallas_call(
    _lse_body,
    grid=(_NB,),
    in_specs=[
        pl.BlockSpec((_BATCH, _DIM), lambda i: (0, 0)),
        pl.BlockSpec((_C_BLK, _DIM), lambda i: (i, 0)),
        pl.BlockSpec((_BATCH, _DIM), lambda i: (0, 0)),
    ],
    out_specs=pl.BlockSpec(memory_space=pltpu.SMEM),
    out_shape=jax.ShapeDtypeStruct((1, 1), jnp.float32),
    scratch_shapes=[
        pltpu.VMEM((_BATCH, 1), jnp.float32),
        pltpu.VMEM((_BATCH, 1), jnp.float32),
    ],
)


def kernel(features, targets, cluster_memory):
    tgt_rows = _sc_gather(cluster_memory, targets)
    loss = _lse_call(features, cluster_memory, tgt_rows)
    return loss[0, 0]


# speedup vs baseline: 2.3918x; 2.3918x over previous
import functools

import jax
import jax.numpy as jnp
from jax import lax
from jax.experimental import pallas as pl
from jax.experimental.pallas import tpu as pltpu
from jax.experimental.pallas import tpu_sc as plsc

_TEMP = 0.05
_INV_TEMP = 1.0 / _TEMP
_NUM_CLUSTERS = 100000
_DIM = 64
_BATCH = 1024

_C_BLK = 5000
_NB = _NUM_CLUSTERS // _C_BLK


@functools.cache
def _make_sc_gather():
    nc, ns = 2, 16
    nw = nc * ns
    b_per_w = _BATCH // nw
    mesh = plsc.VectorSubcoreMesh(core_axis_name="c", subcore_axis_name="s")

    @functools.partial(
        pl.kernel,
        mesh=mesh,
        out_type=jax.ShapeDtypeStruct((_BATCH, _DIM), jnp.float32),
        scratch_types=[
            pltpu.VMEM((b_per_w,), jnp.int32),
            pltpu.VMEM((b_per_w, _DIM), jnp.float32),
            pltpu.SemaphoreType.DMA,
        ],
        compiler_params=pltpu.CompilerParams(use_tc_tiling_on_sc=False),
    )
    def gather_rows(table_hbm, idx_hbm, out_hbm, idx_v, rows_v, sem):
        wid = lax.axis_index("s") * nc + lax.axis_index("c")
        base = wid * b_per_w
        pltpu.sync_copy(idx_hbm.at[pl.ds(base, b_per_w)], idx_v)
        pltpu.async_copy(table_hbm.at[idx_v], rows_v, sem).wait()
        pltpu.sync_copy(rows_v, out_hbm.at[pl.ds(base, b_per_w)])

    return gather_rows


def _lse_body(f_ref, mem_ref, tgt_ref, out_ref, m_ref, s_ref):
    i = pl.program_id(0)

    @pl.when(i == 0)
    def _init():
        m_ref[...] = jnp.full((_BATCH, 1), -1e30, jnp.float32)
        s_ref[...] = jnp.zeros((_BATCH, 1), jnp.float32)

    f = f_ref[...] * _INV_TEMP
    scores = lax.dot_general(
        f, mem_ref[...],
        (((1,), (1,)), ((), ())),
        preferred_element_type=jnp.float32,
    )
    bm = jnp.max(scores, axis=1, keepdims=True)
    m_prev = m_ref[...]
    s_prev = s_ref[...]
    m_new = jnp.maximum(m_prev, bm)
    s_new = s_prev * jnp.exp(m_prev - m_new) + jnp.sum(
        jnp.exp(scores - m_new), axis=1, keepdims=True)
    m_ref[...] = m_new
    s_ref[...] = s_new

    @pl.when(i == _NB - 1)
    def _final():
        tl = jnp.sum(f * tgt_ref[...], axis=1, keepdims=True)
        nll = m_new + jnp.log(s_new) - tl
        out_ref[0, 0] = jnp.mean(nll)


_lse_call = pl.pallas_call(
    _lse_body,
    grid=(_NB,),
    in_specs=[
        pl.BlockSpec((_BATCH, _DIM), lambda i: (0, 0)),
        pl.BlockSpec((_C_BLK, _DIM), lambda i: (i, 0)),
        pl.BlockSpec((_BATCH, _DIM), lambda i: (0, 0)),
    ],
    out_specs=pl.BlockSpec(memory_space=pltpu.SMEM),
    out_shape=jax.ShapeDtypeStruct((1, 1), jnp.float32),
    scratch_shapes=[
        pltpu.VMEM((_BATCH, 1), jnp.float32),
        pltpu.VMEM((_BATCH, 1), jnp.float32),
    ],
)


def kernel(features, targets, cluster_memory):
    tgt_rows = _make_sc_gather()(cluster_memory, targets)
    loss = _lse_call(features, cluster_memory, tgt_rows)
    return loss[0, 0]


# speedup vs baseline: 2.4575x; 1.0275x over previous
import functools

import jax
import jax.numpy as jnp
from jax import lax
from jax.experimental import pallas as pl
from jax.experimental.pallas import tpu as pltpu
from jax.experimental.pallas import tpu_sc as plsc

_TEMP = 0.05
_INV_TEMP = 1.0 / _TEMP
_NUM_CLUSTERS = 100000
_DIM = 64
_BATCH = 1024

_C_BLK = 5000
_NB = _NUM_CLUSTERS // _C_BLK


@functools.cache
def _make_sc_gather():
    nc, ns = 2, 16
    nw = nc * ns
    b_per_w = _BATCH // nw
    mesh = plsc.VectorSubcoreMesh(core_axis_name="c", subcore_axis_name="s")

    @functools.partial(
        pl.kernel,
        mesh=mesh,
        out_type=jax.ShapeDtypeStruct((_BATCH, _DIM), jnp.float32),
        scratch_types=[
            pltpu.VMEM((b_per_w,), jnp.int32),
            pltpu.VMEM((b_per_w, _DIM), jnp.float32),
            pltpu.SemaphoreType.DMA,
        ],
        compiler_params=pltpu.CompilerParams(use_tc_tiling_on_sc=False),
    )
    def gather_rows(table_hbm, idx_hbm, out_hbm, idx_v, rows_v, sem):
        wid = lax.axis_index("s") * nc + lax.axis_index("c")
        base = wid * b_per_w
        pltpu.sync_copy(idx_hbm.at[pl.ds(base, b_per_w)], idx_v)
        pltpu.async_copy(table_hbm.at[idx_v], rows_v, sem).wait()
        pltpu.sync_copy(rows_v, out_hbm.at[pl.ds(base, b_per_w)])

    return gather_rows


_LOG2E = 1.4426950408889634
_LN2 = 0.6931471805599453


def _lse_body(f_ref, mem_ref, tgt_ref, out_ref, m_ref, s_ref):
    i = pl.program_id(0)

    @pl.when(i == 0)
    def _init():
        m_ref[...] = jnp.full((_BATCH, 1), -1e30, jnp.float32)
        s_ref[...] = jnp.zeros((_BATCH, 1), jnp.float32)

    f = f_ref[...] * (_INV_TEMP * _LOG2E)
    scores = lax.dot_general(
        f.astype(jnp.bfloat16), mem_ref[...].astype(jnp.bfloat16),
        (((1,), (1,)), ((), ())),
        preferred_element_type=jnp.float32,
    )
    bm = jnp.max(scores, axis=1, keepdims=True)
    m_prev = m_ref[...]
    s_prev = s_ref[...]
    m_new = jnp.maximum(m_prev, bm)
    s_new = s_prev * jnp.exp2(m_prev - m_new) + jnp.sum(
        jnp.exp2(scores - m_new), axis=1, keepdims=True)
    m_ref[...] = m_new
    s_ref[...] = s_new

    @pl.when(i == _NB - 1)
    def _final():
        tl = jnp.sum(f * tgt_ref[...], axis=1, keepdims=True)
        nll = (m_new + jnp.log2(s_new) - tl) * _LN2
        out_ref[0, 0] = jnp.mean(nll)


_lse_call = pl.pallas_call(
    _lse_body,
    grid=(_NB,),
    in_specs=[
        pl.BlockSpec((_BATCH, _DIM), lambda i: (0, 0)),
        pl.BlockSpec((_C_BLK, _DIM), lambda i: (i, 0)),
        pl.BlockSpec((_BATCH, _DIM), lambda i: (0, 0)),
    ],
    out_specs=pl.BlockSpec(memory_space=pltpu.SMEM),
    out_shape=jax.ShapeDtypeStruct((1, 1), jnp.float32),
    scratch_shapes=[
        pltpu.VMEM((_BATCH, 1), jnp.float32),
        pltpu.VMEM((_BATCH, 1), jnp.float32),
    ],
)


def kernel(features, targets, cluster_memory):
    tgt_rows = _make_sc_gather()(cluster_memory, targets)
    loss = _lse_call(features, cluster_memory, tgt_rows)
    return loss[0, 0]


# speedup vs baseline: 3.2272x; 1.3132x over previous
import functools

import jax
import jax.numpy as jnp
from jax import lax
from jax.experimental import pallas as pl
from jax.experimental.pallas import tpu as pltpu
from jax.experimental.pallas import tpu_sc as plsc

_TEMP = 0.05
_INV_TEMP = 1.0 / _TEMP
_NUM_CLUSTERS = 100000
_DIM = 64
_BATCH = 1024

_C_BLK = 5000
_NB = _NUM_CLUSTERS // _C_BLK


@functools.cache
def _make_sc_gather():
    nc, ns = 2, 16
    nw = nc * ns
    b_per_w = _BATCH // nw
    mesh = plsc.VectorSubcoreMesh(core_axis_name="c", subcore_axis_name="s")

    @functools.partial(
        pl.kernel,
        mesh=mesh,
        out_type=jax.ShapeDtypeStruct((_BATCH, _DIM), jnp.float32),
        scratch_types=[
            pltpu.VMEM((b_per_w,), jnp.int32),
            pltpu.VMEM((b_per_w, _DIM), jnp.float32),
            pltpu.SemaphoreType.DMA,
        ],
        compiler_params=pltpu.CompilerParams(use_tc_tiling_on_sc=False),
    )
    def gather_rows(table_hbm, idx_hbm, out_hbm, idx_v, rows_v, sem):
        wid = lax.axis_index("s") * nc + lax.axis_index("c")
        base = wid * b_per_w
        pltpu.sync_copy(idx_hbm.at[pl.ds(base, b_per_w)], idx_v)
        pltpu.async_copy(table_hbm.at[idx_v], rows_v, sem).wait()
        pltpu.sync_copy(rows_v, out_hbm.at[pl.ds(base, b_per_w)])

    return gather_rows


_LOG2E = 1.4426950408889634
_LN2 = 0.6931471805599453


def _lse_body(f_ref, mem_ref, tgt_ref, out_ref, s_ref):
    i = pl.program_id(0)

    @pl.when(i == 0)
    def _init():
        s_ref[...] = jnp.zeros((_BATCH, 1), jnp.float32)

    f = f_ref[...] * (_INV_TEMP * _LOG2E)
    bound = jnp.sqrt(jnp.sum(f * f, axis=1, keepdims=True)) - 100.0
    scores = lax.dot_general(
        f.astype(jnp.bfloat16), mem_ref[...].astype(jnp.bfloat16),
        (((1,), (1,)), ((), ())),
        preferred_element_type=jnp.float32,
    )
    s_ref[...] += jnp.sum(jnp.exp2(scores - bound), axis=1, keepdims=True)

    @pl.when(i == _NB - 1)
    def _final():
        tl = jnp.sum(f * tgt_ref[...], axis=1, keepdims=True)
        s_safe = jnp.maximum(s_ref[...], 1e-37)
        nll = (bound + jnp.log2(s_safe) - tl) * _LN2
        out_ref[0, 0] = jnp.mean(nll)


_lse_call = pl.pallas_call(
    _lse_body,
    grid=(_NB,),
    in_specs=[
        pl.BlockSpec((_BATCH, _DIM), lambda i: (0, 0)),
        pl.BlockSpec((_C_BLK, _DIM), lambda i: (i, 0)),
        pl.BlockSpec((_BATCH, _DIM), lambda i: (0, 0)),
    ],
    out_specs=pl.BlockSpec(memory_space=pltpu.SMEM),
    out_shape=jax.ShapeDtypeStruct((1, 1), jnp.float32),
    scratch_shapes=[
        pltpu.VMEM((_BATCH, 1), jnp.float32),
    ],
)


def kernel(features, targets, cluster_memory):
    tgt_rows = _make_sc_gather()(cluster_memory, targets)
    loss = _lse_call(features, cluster_memory, tgt_rows)
    return loss[0, 0]
